# baseline (speedup 1.0000x reference)
import functools

import jax
import jax.numpy as jnp
from jax import lax
from jax.experimental import pallas as pl
from jax.experimental.pallas import tpu as pltpu
from jax.experimental.pallas import tpu_sc as plsc

NEG_SLOPE = 0.2
NC = 2
NS = 16
NT = NC * NS
CH = 64
LANES = 16


def _round_up(v, m):
    return (v + m - 1) // m * m


def _vgather16(v, idx):
    dnums = lax.GatherDimensionNumbers(
        offset_dims=(), collapsed_slice_dims=(0,), start_index_map=(0,))
    return lax.gather(v, idx[:, None], dnums, (1,),
                      mode=lax.GatherScatterMode.PROMISE_IN_BOUNDS)


def _vbcast(v, lane):
    return _vgather16(v, jnp.full((LANES,), lane, jnp.int32))


def _tc_prep1(xp, W1, asrc_m, adst_m, perm, Np):
    R = 512
    f32 = jnp.float32

    def body(x_ref, w_ref, as_ref, ad_ref, p_ref, tabh_ref, att_ref):
        h = jnp.dot(x_ref[...], w_ref[...], preferred_element_type=f32)
        asrc = jnp.dot(h, as_ref[...], preferred_element_type=f32)
        adst = jnp.dot(h, ad_ref[...], preferred_element_type=f32)
        hp = jnp.dot(h, p_ref[...], preferred_element_type=f32)
        tabh_ref[...] = hp.astype(jnp.bfloat16)
        att_ref[...] = jnp.concatenate([asrc, adst], axis=1)

    grid = ((Np + R - 1) // R,)
    return pl.pallas_call(
        body,
        grid=grid,
        in_specs=[
            pl.BlockSpec((R, 128), lambda i: (i, 0)),
            pl.BlockSpec((128, 128), lambda i: (0, 0)),
            pl.BlockSpec((128, 8), lambda i: (0, 0)),
            pl.BlockSpec((128, 8), lambda i: (0, 0)),
            pl.BlockSpec((128, 128), lambda i: (0, 0)),
        ],
        out_specs=[
            pl.BlockSpec((R, 128), lambda i: (i, 0)),
            pl.BlockSpec((R, 16), lambda i: (i, 0)),
        ],
        out_shape=[
            jax.ShapeDtypeStruct((Np, 128), jnp.bfloat16),
            jax.ShapeDtypeStruct((Np, 16), f32),
        ],
    )(xp, W1, asrc_m, adst_m, perm)


def _sc_layer1(Np, Epad):
    per_tile = Epad // NT
    n_chunks = per_tile // CH
    rows_per_tile = Np // NS
    stripe_chunks = [(j * CH, CH) for j in range(rows_per_tile // CH)]
    if rows_per_tile % CH:
        stripe_chunks.append((rows_per_tile // CH * CH, rows_per_tile % CH))
    f32 = jnp.float32
    mesh = plsc.VectorSubcoreMesh(
        core_axis_name="c", subcore_axis_name="s", num_cores=NC,
        num_subcores=NS)

    assert n_chunks % 2 == 0

    @functools.partial(
        pl.kernel,
        out_type=jax.ShapeDtypeStruct((NC, Np, 144), f32),
        mesh=mesh,
        compiler_params=pltpu.CompilerParams(use_tc_tiling_on_sc=False, needs_layout_passes=False),
        scratch_types=[
            [pltpu.VMEM((CH,), jnp.int32)] * 2,
            [pltpu.VMEM((CH,), jnp.int32)] * 2,
            [pltpu.VMEM((CH,), jnp.int32)] * 2,
            [pltpu.VMEM((CH, 128), jnp.bfloat16)] * 2,
            [pltpu.VMEM((CH, 16), f32)] * 2,
            [pltpu.VMEM((CH, 16), f32)] * 2,
            [pltpu.VMEM((CH, 144), f32)] * 2,
            pltpu.VMEM_SHARED((Np, 144), f32),
            [pltpu.SemaphoreType.DMA] * 2,
            [pltpu.SemaphoreType.DMA] * 2,
            [pltpu.SemaphoreType.DMA] * 2,
            [pltpu.SemaphoreType.DMA] * 2,
            [pltpu.SemaphoreType.DMA] * 2,
        ],
    )
    def sc1(src_hbm, dst_hbm, tabh_hbm, att_hbm, out_hbm,
            srcv, dstv, dsts, rowsh, atts, attd, msg, acc,
            isem, gsh, gsa, gsd, ssem):
        cid = lax.axis_index("c")
        sid = lax.axis_index("s")
        wid = sid * NC + cid
        ebase = wid * per_tile

        def start_idx(k, b, sync):
            sl = pl.ds(ebase + k * CH, CH)
            if sync:
                pltpu.sync_copy(src_hbm.at[sl], srcv[b])
                pltpu.sync_copy(dst_hbm.at[sl], dstv[b])
            else:
                pltpu.async_copy(src_hbm.at[sl], srcv[b], isem[b])
                pltpu.async_copy(dst_hbm.at[sl], dstv[b], isem[b])

        def wait_idx(b):
            pltpu.make_async_copy(
                src_hbm.at[pl.ds(0, CH)], srcv[b], isem[b]).wait()
            pltpu.make_async_copy(
                dst_hbm.at[pl.ds(0, CH)], dstv[b], isem[b]).wait()

        def start_gather(b):
            pltpu.async_copy(tabh_hbm.at[srcv[b]], rowsh[b], gsh[b])
            pltpu.async_copy(att_hbm.at[srcv[b]], atts[b], gsa[b])
            pltpu.async_copy(att_hbm.at[dstv[b]], attd[b], gsd[b])

        def wait_gather(b):
            pltpu.make_async_copy(
                tabh_hbm.at[srcv[b]], rowsh[b], gsh[b]).wait()
            pltpu.make_async_copy(
                att_hbm.at[srcv[b]], atts[b], gsa[b]).wait()
            pltpu.make_async_copy(
                att_hbm.at[dstv[b]], attd[b], gsd[b]).wait()

        def wait_scatter(b):
            pltpu.make_async_copy(
                msg[b], acc.at[dsts[b]], ssem[b]).wait()

        start_idx(0, 0, True)
        start_gather(0)
        start_idx(1, 1, False)

        zero16 = jnp.zeros((LANES,), f32)

        def zrow(i, carry):
            for j in range(144 // LANES):
                msg[0][i, pl.ds(j * LANES, LANES)] = zero16
            return carry

        lax.fori_loop(0, CH, zrow, 0)
        for off, ln in stripe_chunks:
            pltpu.sync_copy(
                msg[0].at[pl.ds(0, ln)],
                acc.at[pl.ds(sid * rows_per_tile + off, ln)])
        plsc.subcore_barrier()

        def pair(p, carry):
            for b in (0, 1):
                ob = 1 - b
                k = 2 * p + b

                @pl.when(k + 1 < n_chunks)
                def _():
                    wait_idx(ob)
                    start_gather(ob)

                wait_gather(b)

                @pl.when(k >= 2)
                def _():
                    wait_scatter(b)

                for j in range(CH // LANES):
                    dsts[b][pl.ds(j * LANES, LANES)] = (
                        dstv[b][pl.ds(j * LANES, LANES)])

                @pl.when(k + 2 < n_chunks)
                def _():
                    start_idx(k + 2, b, False)

                rb, sb, db, mb = rowsh[b], atts[b], attd[b], msg[b]
                iota16 = lax.iota(jnp.int32, LANES)
                rot8 = jnp.where(iota16 < 8, iota16 + 8, iota16)

                def _edge(q, ecarry):
                    for u in range(2):
                        c = 2 * q + u
                        e16 = sb[c, :] + _vgather16(db[c, :], rot8)
                        e16 = jnp.where(e16 > 0.0, e16, NEG_SLOPE * e16)
                        w16 = jnp.exp(e16)
                        mb[c, pl.ds(128, LANES)] = w16
                        for p in range(4):
                            hpair = rb[c, pl.ds(32 * p, 2 * LANES)]
                            ha, hb2 = plsc.unpack(
                                hpair, format=plsc.PackFormat.INTERLEAVED)
                            mb[c, pl.ds((2 * p) * LANES, LANES)] = (
                                _vbcast(w16, 2 * p) * ha)
                            mb[c, pl.ds((2 * p + 1) * LANES, LANES)] = (
                                _vbcast(w16, 2 * p + 1) * hb2)
                    return ecarry

                lax.fori_loop(0, CH // 2, _edge, 0)

                pltpu.async_copy(mb, acc.at[dsts[b]], ssem[b], add=True)
            return carry

        lax.fori_loop(0, n_chunks // 2, pair, 0)
        wait_scatter(0)
        wait_scatter(1)
        plsc.subcore_barrier()
        for off, ln in stripe_chunks:
            sl = pl.ds(sid * rows_per_tile + off, ln)
            pltpu.sync_copy(acc.at[sl], out_hbm.at[cid, sl])

    return sc1


def _tc_mid(acc1, b1, W2, asv2, adv2, e8, Np):
    R = 512
    f32 = jnp.float32

    def body(acc_ref, b1_ref, w2_ref, as_ref, ad_ref, e8_ref,
             tab_ref, adst_ref):
        a = acc_ref[0] + acc_ref[1]
        num = a[:, :128]
        den8 = a[:, 128:136]
        den = jnp.dot(den8, e8_ref[...], preferred_element_type=f32)
        pre = num / den + b1_ref[...]
        out1 = jnp.where(pre > 0.0, pre, jnp.exp(pre) - 1.0)
        h2 = jnp.dot(out1, w2_ref[...], preferred_element_type=f32)
        asrc2 = jnp.sum(h2 * as_ref[...], axis=1, keepdims=True)
        adst2 = jnp.sum(h2 * ad_ref[...], axis=1, keepdims=True)
        pad = jnp.zeros((R, 14), f32)
        tab_ref[...] = jnp.concatenate([h2, asrc2, adst2, pad], axis=1)
        adst_ref[...] = jnp.concatenate(
            [adst2, jnp.zeros((R, 7), f32)], axis=1)

    grid = ((Np + R - 1) // R,)
    return pl.pallas_call(
        body,
        grid=grid,
        in_specs=[
            pl.BlockSpec((2, R, 144), lambda i: (0, i, 0)),
            pl.BlockSpec((1, 128), lambda i: (0, 0)),
            pl.BlockSpec((128, 16), lambda i: (0, 0)),
            pl.BlockSpec((1, 16), lambda i: (0, 0)),
            pl.BlockSpec((1, 16), lambda i: (0, 0)),
            pl.BlockSpec((8, 128), lambda i: (0, 0)),
        ],
        out_specs=[
            pl.BlockSpec((R, 32), lambda i: (i, 0)),
            pl.BlockSpec((R, 8), lambda i: (i, 0)),
        ],
        out_shape=[
            jax.ShapeDtypeStruct((Np, 32), f32),
            jax.ShapeDtypeStruct((Np, 8), f32),
        ],
    )(acc1, b1, W2, asv2, adv2, e8)


def _sc_layer2(Np, Epad):
    per_tile = Epad // NT
    n_chunks = per_tile // CH
    rows_per_tile = Np // NS
    stripe_chunks = [(j * CH, CH) for j in range(rows_per_tile // CH)]
    if rows_per_tile % CH:
        stripe_chunks.append((rows_per_tile // CH * CH, rows_per_tile % CH))
    f32 = jnp.float32
    mesh = plsc.VectorSubcoreMesh(
        core_axis_name="c", subcore_axis_name="s", num_cores=NC,
        num_subcores=NS)

    assert n_chunks % 2 == 0

    @functools.partial(
        pl.kernel,
        out_type=jax.ShapeDtypeStruct((NC, Np, 32), f32),
        mesh=mesh,
        compiler_params=pltpu.CompilerParams(use_tc_tiling_on_sc=False, needs_layout_passes=False),
        scratch_types=[
            [pltpu.VMEM((CH,), jnp.int32)] * 2,
            [pltpu.VMEM((CH,), jnp.int32)] * 2,
            [pltpu.VMEM((CH,), jnp.int32)] * 2,
            [pltpu.VMEM((CH, 32), f32)] * 2,
            [pltpu.VMEM((CH, 8), f32)] * 2,
            [pltpu.VMEM((CH, 32), f32)] * 2,
            pltpu.VMEM_SHARED((Np, 32), f32),
            [pltpu.SemaphoreType.DMA] * 2,
            [pltpu.SemaphoreType.DMA] * 2,
            [pltpu.SemaphoreType.DMA] * 2,
            [pltpu.SemaphoreType.DMA] * 2,
        ],
    )
    def sc2(src_hbm, dst_hbm, tab_hbm, adst_hbm, out_hbm,
            srcv, dstv, dsts, rows, adv, msg, acc, isem, gsr, gsa, ssem):
        cid = lax.axis_index("c")
        sid = lax.axis_index("s")
        wid = sid * NC + cid
        ebase = wid * per_tile

        def start_idx(k, b, sync):
            sl = pl.ds(ebase + k * CH, CH)
            if sync:
                pltpu.sync_copy(src_hbm.at[sl], srcv[b])
                pltpu.sync_copy(dst_hbm.at[sl], dstv[b])
            else:
                pltpu.async_copy(src_hbm.at[sl], srcv[b], isem[b])
                pltpu.async_copy(dst_hbm.at[sl], dstv[b], isem[b])

        def wait_idx(b):
            pltpu.make_async_copy(
                src_hbm.at[pl.ds(0, CH)], srcv[b], isem[b]).wait()
            pltpu.make_async_copy(
                dst_hbm.at[pl.ds(0, CH)], dstv[b], isem[b]).wait()

        def start_gather(b):
            pltpu.async_copy(tab_hbm.at[srcv[b]], rows[b], gsr[b])
            pltpu.async_copy(adst_hbm.at[dstv[b]], adv[b], gsa[b])

        def wait_gather(b):
            pltpu.make_async_copy(
                tab_hbm.at[srcv[b]], rows[b], gsr[b]).wait()
            pltpu.make_async_copy(
                adst_hbm.at[dstv[b]], adv[b], gsa[b]).wait()

        def wait_scatter(b):
            pltpu.make_async_copy(
                msg[b], acc.at[dsts[b]], ssem[b]).wait()

        start_idx(0, 0, True)
        start_gather(0)
        start_idx(1, 1, False)

        zero16 = jnp.zeros((LANES,), f32)

        def zrow(i, carry):
            msg[0][i, pl.ds(0, LANES)] = zero16
            msg[0][i, pl.ds(LANES, LANES)] = zero16
            return carry

        lax.fori_loop(0, CH, zrow, 0)
        for off, ln in stripe_chunks:
            pltpu.sync_copy(
                msg[0].at[pl.ds(0, ln)],
                acc.at[pl.ds(sid * rows_per_tile + off, ln)])
        plsc.subcore_barrier()

        iota16 = lax.iota(jnp.int32, LANES)
        col_as = jnp.full((LANES,), 16, jnp.int32)
        col_ad = jnp.zeros((LANES,), jnp.int32)

        def pair(p, carry):
            for b in (0, 1):
                ob = 1 - b
                k = 2 * p + b

                @pl.when(k + 1 < n_chunks)
                def _():
                    wait_idx(ob)
                    start_gather(ob)

                wait_gather(b)

                @pl.when(k >= 2)
                def _():
                    wait_scatter(b)

                for j in range(CH // LANES):
                    dsts[b][pl.ds(j * LANES, LANES)] = (
                        dstv[b][pl.ds(j * LANES, LANES)])

                @pl.when(k + 2 < n_chunks)
                def _():
                    start_idx(k + 2, b, False)

                rb, ab, mb = rows[b], adv[b], msg[b]
                onehot0 = jnp.where(iota16 == 0, 1.0, 0.0).astype(f32)

                def group(g, gcarry):
                    c_idx = g * LANES + iota16
                    asrc16 = plsc.load_gather(rb, [c_idx, col_as])
                    adst16 = plsc.load_gather(ab, [c_idx, col_ad])
                    e16 = asrc16 + adst16
                    e16 = jnp.where(e16 > 0.0, e16, NEG_SLOPE * e16)
                    w16 = jnp.exp(e16)
                    for l in range(LANES):
                        c = g * LANES + l
                        wb = _vbcast(w16, l)
                        mb[c, pl.ds(0, LANES)] = wb * rb[c, pl.ds(0, LANES)]
                        mb[c, pl.ds(LANES, LANES)] = wb * onehot0
                    return gcarry

                lax.fori_loop(0, CH // LANES, group, 0)
                pltpu.async_copy(mb, acc.at[dsts[b]], ssem[b], add=True)
            return carry

        lax.fori_loop(0, n_chunks // 2, pair, 0)
        wait_scatter(0)
        wait_scatter(1)
        plsc.subcore_barrier()
        for off, ln in stripe_chunks:
            sl = pl.ds(sid * rows_per_tile + off, ln)
            pltpu.sync_copy(acc.at[sl], out_hbm.at[cid, sl])

    return sc2


def _tc_final(acc2, b2, Np):
    R = 512
    f32 = jnp.float32

    def body(acc_ref, b2_ref, out_ref):
        a = acc_ref[0] + acc_ref[1]
        num = a[:, :16]
        den = a[:, 16:17]
        out_ref[...] = num / den + b2_ref[...]

    grid = ((Np + R - 1) // R,)
    return pl.pallas_call(
        body,
        grid=grid,
        in_specs=[
            pl.BlockSpec((2, R, 32), lambda i: (0, i, 0)),
            pl.BlockSpec((1, 16), lambda i: (0, 0)),
        ],
        out_specs=pl.BlockSpec((R, 16), lambda i: (i, 0)),
        out_shape=jax.ShapeDtypeStruct((Np, 16), f32),
    )(acc2, b2)


def kernel(x, edge_index, W1, a_src1, a_dst1, b1, W2, a_src2, a_dst2, b2):
    N, DIN = x.shape
    E = edge_index.shape[1]
    H1, F1 = a_src1.shape
    f32 = jnp.float32

    Np = _round_up(N + 1, NS)
    Etot = E + N
    Epad = _round_up(Etot, NT * CH * 2)

    loop = jnp.arange(N, dtype=edge_index.dtype)
    padv = jnp.full((Epad - Etot,), Np - 1, edge_index.dtype)
    src = jnp.concatenate([edge_index[0], loop, padv])
    dst = jnp.concatenate([edge_index[1], loop, padv])

    xp = jnp.zeros((Np, DIN), f32).at[:N].set(x)
    eye8 = jnp.eye(H1, dtype=f32)
    asrc_m = (eye8[:, None, :] * a_src1[:, :, None]).reshape(H1 * F1, H1)
    adst_m = (eye8[:, None, :] * a_dst1[:, :, None]).reshape(H1 * F1, H1)
    e8 = jnp.kron(jnp.eye(H1, dtype=f32), jnp.ones((1, F1), f32))

    order = [0] * (H1 * F1)
    for p in range(H1 // 2):
        for i in range(F1):
            for j in range(2):
                order[32 * p + 2 * i + j] = (2 * p + j) * F1 + i
    perm = jnp.eye(H1 * F1, dtype=f32)[jnp.asarray(order)].T

    tabh1, att1 = _tc_prep1(xp, W1, asrc_m, adst_m, perm, Np)
    acc1 = _sc_layer1(Np, Epad)(src, dst, tabh1, att1)

    tab2, adst2 = _tc_mid(acc1, b1.reshape(1, -1), W2,
                          a_src2.reshape(1, -1), a_dst2.reshape(1, -1),
                          e8, Np)

    acc2 = _sc_layer2(Np, Epad)(src, dst, tab2, adst2)

    out = _tc_final(acc2, b2.reshape(1, -1), Np)
    return out[:N]


# speedup vs baseline: 1.2062x; 1.2062x over previous
import functools

import jax
import jax.numpy as jnp
from jax import lax
from jax.experimental import pallas as pl
from jax.experimental.pallas import tpu as pltpu
from jax.experimental.pallas import tpu_sc as plsc

NEG_SLOPE = 0.2
NC = 2
NS = 16
NT = NC * NS
CH = 64
LANES = 16


def _round_up(v, m):
    return (v + m - 1) // m * m


def _vgather16(v, idx):
    dnums = lax.GatherDimensionNumbers(
        offset_dims=(), collapsed_slice_dims=(0,), start_index_map=(0,))
    return lax.gather(v, idx[:, None], dnums, (1,),
                      mode=lax.GatherScatterMode.PROMISE_IN_BOUNDS)


def _vbcast(v, lane):
    return _vgather16(v, jnp.full((LANES,), lane, jnp.int32))


def _tc_prep1(xp, W1, asrc_m, adst_m, perm, Np):
    R = 512
    f32 = jnp.float32

    def body(x_ref, w_ref, as_ref, ad_ref, p_ref, tab_ref, adst_ref):
        h = jnp.dot(x_ref[...], w_ref[...], preferred_element_type=f32)
        asrc = jnp.dot(h, as_ref[...], preferred_element_type=f32)
        adst = jnp.dot(h, ad_ref[...], preferred_element_type=f32)
        tab_ref[...] = jnp.concatenate([h, asrc, adst], axis=1)
        adst_ref[...] = jnp.concatenate(
            [adst, jnp.zeros((R, 8), f32)], axis=1)

    grid = ((Np + R - 1) // R,)
    return pl.pallas_call(
        body,
        grid=grid,
        in_specs=[
            pl.BlockSpec((R, 128), lambda i: (i, 0)),
            pl.BlockSpec((128, 128), lambda i: (0, 0)),
            pl.BlockSpec((128, 8), lambda i: (0, 0)),
            pl.BlockSpec((128, 8), lambda i: (0, 0)),
            pl.BlockSpec((128, 128), lambda i: (0, 0)),
        ],
        out_specs=[
            pl.BlockSpec((R, 144), lambda i: (i, 0)),
            pl.BlockSpec((R, 16), lambda i: (i, 0)),
        ],
        out_shape=[
            jax.ShapeDtypeStruct((Np, 144), f32),
            jax.ShapeDtypeStruct((Np, 16), f32),
        ],
    )(xp, W1, asrc_m, adst_m, perm)


def _sc_layer1(Np, Epad):
    per_tile = Epad // NT
    n_chunks = per_tile // CH
    rows_per_tile = Np // NS
    stripe_chunks = [(j * CH, CH) for j in range(rows_per_tile // CH)]
    if rows_per_tile % CH:
        stripe_chunks.append((rows_per_tile // CH * CH, rows_per_tile % CH))
    f32 = jnp.float32
    mesh = plsc.VectorSubcoreMesh(
        core_axis_name="c", subcore_axis_name="s", num_cores=NC,
        num_subcores=NS)

    assert n_chunks % 2 == 0

    @functools.partial(
        pl.kernel,
        out_type=jax.ShapeDtypeStruct((NC, Np, 144), f32),
        mesh=mesh,
        compiler_params=pltpu.CompilerParams(use_tc_tiling_on_sc=False, needs_layout_passes=False),
        scratch_types=[
            [pltpu.VMEM((2 * CH,), jnp.int32)] * 2,
            [pltpu.VMEM((CH,), jnp.int32)] * 2,
            [pltpu.VMEM((CH,), jnp.int32)] * 2,
            [pltpu.VMEM((CH,), jnp.int32)] * 2,
            [pltpu.VMEM((CH, 144), f32)] * 2,
            [pltpu.VMEM((CH, 16), f32)] * 2,
            [pltpu.VMEM((CH, 144), f32)] * 2,
            pltpu.VMEM_SHARED((Np, 144), f32),
            [pltpu.SemaphoreType.DMA] * 2,
            [pltpu.SemaphoreType.DMA] * 2,
            [pltpu.SemaphoreType.DMA] * 2,
            [pltpu.SemaphoreType.DMA] * 2,
        ],
    )
    def sc1(ei_hbm, tab_hbm, adst_hbm, out_hbm,
            eidx, srcv, dstv, dsts, rows, adv, msg, acc,
            isem, gsr, gsa, ssem):
        cid = lax.axis_index("c")
        sid = lax.axis_index("s")
        wid = sid * NC + cid
        cbase = wid * n_chunks

        def start_idx(k, b, sync):
            if sync:
                pltpu.sync_copy(ei_hbm.at[cbase + k], eidx[b])
            else:
                pltpu.async_copy(ei_hbm.at[cbase + k], eidx[b], isem[b])

        def wait_idx(b):
            pltpu.make_async_copy(
                ei_hbm.at[0], eidx[b], isem[b]).wait()

        def unpack_idx(b):
            for j in range(CH // LANES):
                srcv[b][pl.ds(j * LANES, LANES)] = (
                    eidx[b][pl.ds(j * LANES, LANES)])
                dstv[b][pl.ds(j * LANES, LANES)] = (
                    eidx[b][pl.ds(CH + j * LANES, LANES)])

        def start_gather(b):
            pltpu.async_copy(tab_hbm.at[srcv[b]], rows[b], gsr[b])
            pltpu.async_copy(adst_hbm.at[dstv[b]], adv[b], gsa[b])

        def wait_gather(b):
            pltpu.make_async_copy(
                tab_hbm.at[srcv[b]], rows[b], gsr[b]).wait()
            pltpu.make_async_copy(
                adst_hbm.at[dstv[b]], adv[b], gsa[b]).wait()

        def wait_scatter(b):
            pltpu.make_async_copy(
                msg[b], acc.at[dsts[b]], ssem[b]).wait()

        start_idx(0, 0, True)
        unpack_idx(0)
        start_gather(0)
        start_idx(1, 1, False)

        zero16 = jnp.zeros((LANES,), f32)

        def zrow(i, carry):
            for j in range(144 // LANES):
                msg[0][i, pl.ds(j * LANES, LANES)] = zero16
            return carry

        lax.fori_loop(0, CH, zrow, 0)
        for off, ln in stripe_chunks:
            pltpu.sync_copy(
                msg[0].at[pl.ds(0, ln)],
                acc.at[pl.ds(sid * rows_per_tile + off, ln)])
        plsc.subcore_barrier()

        def pair(p, carry):
            for b in (0, 1):
                ob = 1 - b
                k = 2 * p + b

                @pl.when(k + 1 < n_chunks)
                def _():
                    wait_idx(ob)
                    unpack_idx(ob)
                    start_gather(ob)

                wait_gather(b)

                @pl.when(k >= 2)
                def _():
                    wait_scatter(b)

                for j in range(CH // LANES):
                    dsts[b][pl.ds(j * LANES, LANES)] = (
                        dstv[b][pl.ds(j * LANES, LANES)])

                @pl.when(k + 2 < n_chunks)
                def _():
                    start_idx(k + 2, b, False)

                rb, ab, mb = rows[b], adv[b], msg[b]

                def _edge(q, ecarry):
                    for u in range(2):
                        c = 2 * q + u
                        e16 = rb[c, pl.ds(128, LANES)] + ab[c, :]
                        e16 = jnp.where(e16 > 0.0, e16, NEG_SLOPE * e16)
                        w16 = jnp.exp(e16)
                        mb[c, pl.ds(128, LANES)] = w16
                        for h in range(8):
                            wb = _vbcast(w16, h)
                            mb[c, pl.ds(h * LANES, LANES)] = (
                                wb * rb[c, pl.ds(h * LANES, LANES)])
                    return ecarry

                lax.fori_loop(0, CH // 2, _edge, 0)

                pltpu.async_copy(mb, acc.at[dsts[b]], ssem[b], add=True)
            return carry

        lax.fori_loop(0, n_chunks // 2, pair, 0)
        wait_scatter(0)
        wait_scatter(1)
        plsc.subcore_barrier()
        for off, ln in stripe_chunks:
            sl = pl.ds(sid * rows_per_tile + off, ln)
            pltpu.sync_copy(acc.at[sl], out_hbm.at[cid, sl])

    return sc1


def _tc_mid(acc1, b1, W2, asv2, adv2, e8, Np):
    R = 512
    f32 = jnp.float32

    def body(acc_ref, b1_ref, w2_ref, as_ref, ad_ref, e8_ref,
             tab_ref, adst_ref):
        a = acc_ref[0] + acc_ref[1]
        num = a[:, :128]
        den8 = a[:, 128:136]
        den = jnp.dot(den8, e8_ref[...], preferred_element_type=f32)
        pre = num / den + b1_ref[...]
        out1 = jnp.where(pre > 0.0, pre, jnp.exp(pre) - 1.0)
        h2 = jnp.dot(out1, w2_ref[...], preferred_element_type=f32)
        asrc2 = jnp.sum(h2 * as_ref[...], axis=1, keepdims=True)
        adst2 = jnp.sum(h2 * ad_ref[...], axis=1, keepdims=True)
        pad = jnp.zeros((R, 14), f32)
        tab_ref[...] = jnp.concatenate([h2, asrc2, adst2, pad], axis=1)
        adst_ref[...] = jnp.concatenate(
            [adst2, jnp.zeros((R, 7), f32)], axis=1)

    grid = ((Np + R - 1) // R,)
    return pl.pallas_call(
        body,
        grid=grid,
        in_specs=[
            pl.BlockSpec((2, R, 144), lambda i: (0, i, 0)),
            pl.BlockSpec((1, 128), lambda i: (0, 0)),
            pl.BlockSpec((128, 16), lambda i: (0, 0)),
            pl.BlockSpec((1, 16), lambda i: (0, 0)),
            pl.BlockSpec((1, 16), lambda i: (0, 0)),
            pl.BlockSpec((8, 128), lambda i: (0, 0)),
        ],
        out_specs=[
            pl.BlockSpec((R, 32), lambda i: (i, 0)),
            pl.BlockSpec((R, 8), lambda i: (i, 0)),
        ],
        out_shape=[
            jax.ShapeDtypeStruct((Np, 32), f32),
            jax.ShapeDtypeStruct((Np, 8), f32),
        ],
    )(acc1, b1, W2, asv2, adv2, e8)


def _sc_layer2(Np, Epad):
    per_tile = Epad // NT
    n_chunks = per_tile // CH
    rows_per_tile = Np // NS
    stripe_chunks = [(j * CH, CH) for j in range(rows_per_tile // CH)]
    if rows_per_tile % CH:
        stripe_chunks.append((rows_per_tile // CH * CH, rows_per_tile % CH))
    f32 = jnp.float32
    mesh = plsc.VectorSubcoreMesh(
        core_axis_name="c", subcore_axis_name="s", num_cores=NC,
        num_subcores=NS)

    assert n_chunks % 2 == 0

    @functools.partial(
        pl.kernel,
        out_type=jax.ShapeDtypeStruct((NC, Np, 32), f32),
        mesh=mesh,
        compiler_params=pltpu.CompilerParams(use_tc_tiling_on_sc=False, needs_layout_passes=False),
        scratch_types=[
            [pltpu.VMEM((2 * CH,), jnp.int32)] * 2,
            [pltpu.VMEM((CH,), jnp.int32)] * 2,
            [pltpu.VMEM((CH,), jnp.int32)] * 2,
            [pltpu.VMEM((CH,), jnp.int32)] * 2,
            [pltpu.VMEM((CH, 32), f32)] * 2,
            [pltpu.VMEM((CH, 8), f32)] * 2,
            [pltpu.VMEM((CH, 32), f32)] * 2,
            pltpu.VMEM_SHARED((Np, 32), f32),
            [pltpu.SemaphoreType.DMA] * 2,
            [pltpu.SemaphoreType.DMA] * 2,
            [pltpu.SemaphoreType.DMA] * 2,
            [pltpu.SemaphoreType.DMA] * 2,
        ],
    )
    def sc2(ei_hbm, tab_hbm, adst_hbm, out_hbm,
            eidx, srcv, dstv, dsts, rows, adv, msg, acc,
            isem, gsr, gsa, ssem):
        cid = lax.axis_index("c")
        sid = lax.axis_index("s")
        wid = sid * NC + cid
        cbase = wid * n_chunks

        def start_idx(k, b, sync):
            if sync:
                pltpu.sync_copy(ei_hbm.at[cbase + k], eidx[b])
            else:
                pltpu.async_copy(ei_hbm.at[cbase + k], eidx[b], isem[b])

        def wait_idx(b):
            pltpu.make_async_copy(
                ei_hbm.at[0], eidx[b], isem[b]).wait()

        def unpack_idx(b):
            for j in range(CH // LANES):
                srcv[b][pl.ds(j * LANES, LANES)] = (
                    eidx[b][pl.ds(j * LANES, LANES)])
                dstv[b][pl.ds(j * LANES, LANES)] = (
                    eidx[b][pl.ds(CH + j * LANES, LANES)])

        def start_gather(b):
            pltpu.async_copy(tab_hbm.at[srcv[b]], rows[b], gsr[b])
            pltpu.async_copy(adst_hbm.at[dstv[b]], adv[b], gsa[b])

        def wait_gather(b):
            pltpu.make_async_copy(
                tab_hbm.at[srcv[b]], rows[b], gsr[b]).wait()
            pltpu.make_async_copy(
                adst_hbm.at[dstv[b]], adv[b], gsa[b]).wait()

        def wait_scatter(b):
            pltpu.make_async_copy(
                msg[b], acc.at[dsts[b]], ssem[b]).wait()

        start_idx(0, 0, True)
        unpack_idx(0)
        start_gather(0)
        start_idx(1, 1, False)

        zero16 = jnp.zeros((LANES,), f32)

        def zrow(i, carry):
            msg[0][i, pl.ds(0, LANES)] = zero16
            msg[0][i, pl.ds(LANES, LANES)] = zero16
            return carry

        lax.fori_loop(0, CH, zrow, 0)
        for off, ln in stripe_chunks:
            pltpu.sync_copy(
                msg[0].at[pl.ds(0, ln)],
                acc.at[pl.ds(sid * rows_per_tile + off, ln)])
        plsc.subcore_barrier()

        iota16 = lax.iota(jnp.int32, LANES)
        col_as = jnp.full((LANES,), 16, jnp.int32)
        col_ad = jnp.zeros((LANES,), jnp.int32)

        def pair(p, carry):
            for b in (0, 1):
                ob = 1 - b
                k = 2 * p + b

                @pl.when(k + 1 < n_chunks)
                def _():
                    wait_idx(ob)
                    unpack_idx(ob)
                    start_gather(ob)

                wait_gather(b)

                @pl.when(k >= 2)
                def _():
                    wait_scatter(b)

                for j in range(CH // LANES):
                    dsts[b][pl.ds(j * LANES, LANES)] = (
                        dstv[b][pl.ds(j * LANES, LANES)])

                @pl.when(k + 2 < n_chunks)
                def _():
                    start_idx(k + 2, b, False)

                rb, ab, mb = rows[b], adv[b], msg[b]
                onehot0 = jnp.where(iota16 == 0, 1.0, 0.0).astype(f32)

                def group(g, gcarry):
                    c_idx = g * LANES + iota16
                    asrc16 = plsc.load_gather(rb, [c_idx, col_as])
                    adst16 = plsc.load_gather(ab, [c_idx, col_ad])
                    e16 = asrc16 + adst16
                    e16 = jnp.where(e16 > 0.0, e16, NEG_SLOPE * e16)
                    w16 = jnp.exp(e16)
                    for l in range(LANES):
                        c = g * LANES + l
                        wb = _vbcast(w16, l)
                        mb[c, pl.ds(0, LANES)] = wb * rb[c, pl.ds(0, LANES)]
                        mb[c, pl.ds(LANES, LANES)] = wb * onehot0
                    return gcarry

                lax.fori_loop(0, CH // LANES, group, 0)
                pltpu.async_copy(mb, acc.at[dsts[b]], ssem[b], add=True)
            return carry

        lax.fori_loop(0, n_chunks // 2, pair, 0)
        wait_scatter(0)
        wait_scatter(1)
        plsc.subcore_barrier()
        for off, ln in stripe_chunks:
            sl = pl.ds(sid * rows_per_tile + off, ln)
            pltpu.sync_copy(acc.at[sl], out_hbm.at[cid, sl])

    return sc2


def _tc_final(acc2, b2, Np):
    R = 512
    f32 = jnp.float32

    def body(acc_ref, b2_ref, out_ref):
        a = acc_ref[0] + acc_ref[1]
        num = a[:, :16]
        den = a[:, 16:17]
        out_ref[...] = num / den + b2_ref[...]

    grid = ((Np + R - 1) // R,)
    return pl.pallas_call(
        body,
        grid=grid,
        in_specs=[
            pl.BlockSpec((2, R, 32), lambda i: (0, i, 0)),
            pl.BlockSpec((1, 16), lambda i: (0, 0)),
        ],
        out_specs=pl.BlockSpec((R, 16), lambda i: (i, 0)),
        out_shape=jax.ShapeDtypeStruct((Np, 16), f32),
    )(acc2, b2)


def kernel(x, edge_index, W1, a_src1, a_dst1, b1, W2, a_src2, a_dst2, b2):
    N, DIN = x.shape
    E = edge_index.shape[1]
    H1, F1 = a_src1.shape
    f32 = jnp.float32

    Np = _round_up(N + 1, NS)
    Etot = E + N
    Epad = _round_up(Etot, NT * CH * 2)

    loop = jnp.arange(N, dtype=edge_index.dtype)
    padv = jnp.full((Epad - Etot,), Np - 1, edge_index.dtype)
    src = jnp.concatenate([edge_index[0], loop, padv])
    dst = jnp.concatenate([edge_index[1], loop, padv])
    ei = jnp.concatenate(
        [src.reshape(Epad // CH, CH), dst.reshape(Epad // CH, CH)], axis=1)

    xp = jnp.zeros((Np, DIN), f32).at[:N].set(x)
    eye8 = jnp.eye(H1, dtype=f32)
    asrc_m = (eye8[:, None, :] * a_src1[:, :, None]).reshape(H1 * F1, H1)
    adst_m = (eye8[:, None, :] * a_dst1[:, :, None]).reshape(H1 * F1, H1)
    e8 = jnp.kron(jnp.eye(H1, dtype=f32), jnp.ones((1, F1), f32))
    perm = jnp.eye(H1 * F1, dtype=f32)

    tab1, adst1 = _tc_prep1(xp, W1, asrc_m, adst_m, perm, Np)
    acc1 = _sc_layer1(Np, Epad)(ei, tab1, adst1)

    tab2, adst2 = _tc_mid(acc1, b1.reshape(1, -1), W2,
                          a_src2.reshape(1, -1), a_dst2.reshape(1, -1),
                          e8, Np)

    acc2 = _sc_layer2(Np, Epad)(ei, tab2, adst2)

    out = _tc_final(acc2, b2.reshape(1, -1), Np)
    return out[:N]


# speedup vs baseline: 1.2062x; 1.0000x over previous
import functools

import jax
import jax.numpy as jnp
from jax import lax
from jax.experimental import pallas as pl
from jax.experimental.pallas import tpu as pltpu
from jax.experimental.pallas import tpu_sc as plsc

NEG_SLOPE = 0.2
NC = 2
NS = 16
NT = NC * NS
CH = 64
LANES = 16


def _round_up(v, m):
    return (v + m - 1) // m * m


def _vgather16(v, idx):
    dnums = lax.GatherDimensionNumbers(
        offset_dims=(), collapsed_slice_dims=(0,), start_index_map=(0,))
    return lax.gather(v, idx[:, None], dnums, (1,),
                      mode=lax.GatherScatterMode.PROMISE_IN_BOUNDS)


def _vbcast(v, lane):
    return _vgather16(v, jnp.full((LANES,), lane, jnp.int32))


def _tc_prep1(xp, W1, asrc_m, adst_m, perm, Np):
    R = 512
    f32 = jnp.float32

    def body(x_ref, w_ref, as_ref, ad_ref, p_ref, tab_ref, adst_ref):
        h = jnp.dot(x_ref[...], w_ref[...], preferred_element_type=f32)
        asrc = jnp.dot(h, as_ref[...], preferred_element_type=f32)
        adst = jnp.dot(h, ad_ref[...], preferred_element_type=f32)
        tab_ref[...] = jnp.concatenate([h, asrc, adst], axis=1)
        adst_ref[...] = jnp.concatenate(
            [adst, jnp.zeros((R, 8), f32)], axis=1)

    grid = ((Np + R - 1) // R,)
    return pl.pallas_call(
        body,
        grid=grid,
        in_specs=[
            pl.BlockSpec((R, 128), lambda i: (i, 0)),
            pl.BlockSpec((128, 128), lambda i: (0, 0)),
            pl.BlockSpec((128, 8), lambda i: (0, 0)),
            pl.BlockSpec((128, 8), lambda i: (0, 0)),
            pl.BlockSpec((128, 128), lambda i: (0, 0)),
        ],
        out_specs=[
            pl.BlockSpec((R, 144), lambda i: (i, 0)),
            pl.BlockSpec((R, 16), lambda i: (i, 0)),
        ],
        out_shape=[
            jax.ShapeDtypeStruct((Np, 144), f32),
            jax.ShapeDtypeStruct((Np, 16), f32),
        ],
    )(xp, W1, asrc_m, adst_m, perm)


def _sc_layer1(Np, Epad):
    per_tile = Epad // NT
    n_chunks = per_tile // CH
    rows_per_tile = Np // NS
    stripe_chunks = [(j * CH, CH) for j in range(rows_per_tile // CH)]
    if rows_per_tile % CH:
        stripe_chunks.append((rows_per_tile // CH * CH, rows_per_tile % CH))
    f32 = jnp.float32
    mesh = plsc.VectorSubcoreMesh(
        core_axis_name="c", subcore_axis_name="s", num_cores=NC,
        num_subcores=NS)

    assert n_chunks % 2 == 0

    @functools.partial(
        pl.kernel,
        out_type=jax.ShapeDtypeStruct((NC, Np, 144), f32),
        mesh=mesh,
        compiler_params=pltpu.CompilerParams(use_tc_tiling_on_sc=False, needs_layout_passes=False, skip_device_barrier=True),
        scratch_types=[
            [pltpu.VMEM((2 * CH,), jnp.int32)] * 2,
            [pltpu.VMEM((CH,), jnp.int32)] * 2,
            [pltpu.VMEM((CH,), jnp.int32)] * 2,
            [pltpu.VMEM((CH,), jnp.int32)] * 2,
            [pltpu.VMEM((CH, 144), f32)] * 2,
            [pltpu.VMEM((CH, 16), f32)] * 2,
            [pltpu.VMEM((CH, 144), f32)] * 2,
            pltpu.VMEM_SHARED((Np, 144), f32),
            [pltpu.SemaphoreType.DMA] * 2,
            [pltpu.SemaphoreType.DMA] * 2,
            [pltpu.SemaphoreType.DMA] * 2,
            [pltpu.SemaphoreType.DMA] * 2,
        ],
    )
    def sc1(ei_hbm, tab_hbm, adst_hbm, out_hbm,
            eidx, srcv, dstv, dsts, rows, adv, msg, acc,
            isem, gsr, gsa, ssem):
        cid = lax.axis_index("c")
        sid = lax.axis_index("s")
        wid = sid * NC + cid
        cbase = wid * n_chunks

        def start_idx(k, b, sync):
            if sync:
                pltpu.sync_copy(ei_hbm.at[cbase + k], eidx[b])
            else:
                pltpu.async_copy(ei_hbm.at[cbase + k], eidx[b], isem[b])

        def wait_idx(b):
            pltpu.make_async_copy(
                ei_hbm.at[0], eidx[b], isem[b]).wait()

        def unpack_idx(b):
            for j in range(CH // LANES):
                srcv[b][pl.ds(j * LANES, LANES)] = (
                    eidx[b][pl.ds(j * LANES, LANES)])
                dstv[b][pl.ds(j * LANES, LANES)] = (
                    eidx[b][pl.ds(CH + j * LANES, LANES)])

        def start_gather(b):
            pltpu.async_copy(tab_hbm.at[srcv[b]], rows[b], gsr[b])
            pltpu.async_copy(adst_hbm.at[dstv[b]], adv[b], gsa[b])

        def wait_gather(b):
            pltpu.make_async_copy(
                tab_hbm.at[srcv[b]], rows[b], gsr[b]).wait()
            pltpu.make_async_copy(
                adst_hbm.at[dstv[b]], adv[b], gsa[b]).wait()

        def wait_scatter(b):
            pltpu.make_async_copy(
                msg[b], acc.at[dsts[b]], ssem[b]).wait()

        start_idx(0, 0, True)
        unpack_idx(0)
        start_gather(0)
        start_idx(1, 1, False)

        zero16 = jnp.zeros((LANES,), f32)

        def zrow(i, carry):
            for j in range(144 // LANES):
                msg[0][i, pl.ds(j * LANES, LANES)] = zero16
            return carry

        lax.fori_loop(0, CH, zrow, 0)
        for off, ln in stripe_chunks:
            pltpu.sync_copy(
                msg[0].at[pl.ds(0, ln)],
                acc.at[pl.ds(sid * rows_per_tile + off, ln)])
        plsc.subcore_barrier()

        def pair(p, carry):
            for b in (0, 1):
                ob = 1 - b
                k = 2 * p + b

                @pl.when(k + 1 < n_chunks)
                def _():
                    wait_idx(ob)
                    unpack_idx(ob)
                    start_gather(ob)

                wait_gather(b)

                @pl.when(k >= 2)
                def _():
                    wait_scatter(b)

                for j in range(CH // LANES):
                    dsts[b][pl.ds(j * LANES, LANES)] = (
                        dstv[b][pl.ds(j * LANES, LANES)])

                @pl.when(k + 2 < n_chunks)
                def _():
                    start_idx(k + 2, b, False)

                rb, ab, mb = rows[b], adv[b], msg[b]

                def _edge(q, ecarry):
                    for u in range(2):
                        c = 2 * q + u
                        e16 = rb[c, pl.ds(128, LANES)] + ab[c, :]
                        e16 = jnp.where(e16 > 0.0, e16, NEG_SLOPE * e16)
                        w16 = jnp.exp(e16)
                        mb[c, pl.ds(128, LANES)] = w16
                        for h in range(8):
                            wb = _vbcast(w16, h)
                            mb[c, pl.ds(h * LANES, LANES)] = (
                                wb * rb[c, pl.ds(h * LANES, LANES)])
                    return ecarry

                lax.fori_loop(0, CH // 2, _edge, 0)

                pltpu.async_copy(mb, acc.at[dsts[b]], ssem[b], add=True)
            return carry

        lax.fori_loop(0, n_chunks // 2, pair, 0)
        wait_scatter(0)
        wait_scatter(1)
        plsc.subcore_barrier()
        for off, ln in stripe_chunks:
            sl = pl.ds(sid * rows_per_tile + off, ln)
            pltpu.sync_copy(acc.at[sl], out_hbm.at[cid, sl])

    return sc1


def _tc_mid(acc1, b1, W2, asv2, adv2, e8, Np):
    R = 512
    f32 = jnp.float32

    def body(acc_ref, b1_ref, w2_ref, as_ref, ad_ref, e8_ref,
             tab_ref, adst_ref):
        a = acc_ref[0] + acc_ref[1]
        num = a[:, :128]
        den8 = a[:, 128:136]
        den = jnp.dot(den8, e8_ref[...], preferred_element_type=f32)
        pre = num / den + b1_ref[...]
        out1 = jnp.where(pre > 0.0, pre, jnp.exp(pre) - 1.0)
        h2 = jnp.dot(out1, w2_ref[...], preferred_element_type=f32)
        asrc2 = jnp.sum(h2 * as_ref[...], axis=1, keepdims=True)
        adst2 = jnp.sum(h2 * ad_ref[...], axis=1, keepdims=True)
        pad = jnp.zeros((R, 14), f32)
        tab_ref[...] = jnp.concatenate([h2, asrc2, adst2, pad], axis=1)
        adst_ref[...] = jnp.concatenate(
            [adst2, jnp.zeros((R, 7), f32)], axis=1)

    grid = ((Np + R - 1) // R,)
    return pl.pallas_call(
        body,
        grid=grid,
        in_specs=[
            pl.BlockSpec((2, R, 144), lambda i: (0, i, 0)),
            pl.BlockSpec((1, 128), lambda i: (0, 0)),
            pl.BlockSpec((128, 16), lambda i: (0, 0)),
            pl.BlockSpec((1, 16), lambda i: (0, 0)),
            pl.BlockSpec((1, 16), lambda i: (0, 0)),
            pl.BlockSpec((8, 128), lambda i: (0, 0)),
        ],
        out_specs=[
            pl.BlockSpec((R, 32), lambda i: (i, 0)),
            pl.BlockSpec((R, 8), lambda i: (i, 0)),
        ],
        out_shape=[
            jax.ShapeDtypeStruct((Np, 32), f32),
            jax.ShapeDtypeStruct((Np, 8), f32),
        ],
    )(acc1, b1, W2, asv2, adv2, e8)


def _sc_layer2(Np, Epad):
    per_tile = Epad // NT
    n_chunks = per_tile // CH
    rows_per_tile = Np // NS
    stripe_chunks = [(j * CH, CH) for j in range(rows_per_tile // CH)]
    if rows_per_tile % CH:
        stripe_chunks.append((rows_per_tile // CH * CH, rows_per_tile % CH))
    f32 = jnp.float32
    mesh = plsc.VectorSubcoreMesh(
        core_axis_name="c", subcore_axis_name="s", num_cores=NC,
        num_subcores=NS)

    assert n_chunks % 2 == 0

    @functools.partial(
        pl.kernel,
        out_type=jax.ShapeDtypeStruct((NC, Np, 32), f32),
        mesh=mesh,
        compiler_params=pltpu.CompilerParams(use_tc_tiling_on_sc=False, needs_layout_passes=False, skip_device_barrier=True),
        scratch_types=[
            [pltpu.VMEM((2 * CH,), jnp.int32)] * 2,
            [pltpu.VMEM((CH,), jnp.int32)] * 2,
            [pltpu.VMEM((CH,), jnp.int32)] * 2,
            [pltpu.VMEM((CH,), jnp.int32)] * 2,
            [pltpu.VMEM((CH, 32), f32)] * 2,
            [pltpu.VMEM((CH, 8), f32)] * 2,
            [pltpu.VMEM((CH, 32), f32)] * 2,
            pltpu.VMEM_SHARED((Np, 32), f32),
            [pltpu.SemaphoreType.DMA] * 2,
            [pltpu.SemaphoreType.DMA] * 2,
            [pltpu.SemaphoreType.DMA] * 2,
            [pltpu.SemaphoreType.DMA] * 2,
        ],
    )
    def sc2(ei_hbm, tab_hbm, adst_hbm, out_hbm,
            eidx, srcv, dstv, dsts, rows, adv, msg, acc,
            isem, gsr, gsa, ssem):
        cid = lax.axis_index("c")
        sid = lax.axis_index("s")
        wid = sid * NC + cid
        cbase = wid * n_chunks

        def start_idx(k, b, sync):
            if sync:
                pltpu.sync_copy(ei_hbm.at[cbase + k], eidx[b])
            else:
                pltpu.async_copy(ei_hbm.at[cbase + k], eidx[b], isem[b])

        def wait_idx(b):
            pltpu.make_async_copy(
                ei_hbm.at[0], eidx[b], isem[b]).wait()

        def unpack_idx(b):
            for j in range(CH // LANES):
                srcv[b][pl.ds(j * LANES, LANES)] = (
                    eidx[b][pl.ds(j * LANES, LANES)])
                dstv[b][pl.ds(j * LANES, LANES)] = (
                    eidx[b][pl.ds(CH + j * LANES, LANES)])

        def start_gather(b):
            pltpu.async_copy(tab_hbm.at[srcv[b]], rows[b], gsr[b])
            pltpu.async_copy(adst_hbm.at[dstv[b]], adv[b], gsa[b])

        def wait_gather(b):
            pltpu.make_async_copy(
                tab_hbm.at[srcv[b]], rows[b], gsr[b]).wait()
            pltpu.make_async_copy(
                adst_hbm.at[dstv[b]], adv[b], gsa[b]).wait()

        def wait_scatter(b):
            pltpu.make_async_copy(
                msg[b], acc.at[dsts[b]], ssem[b]).wait()

        start_idx(0, 0, True)
        unpack_idx(0)
        start_gather(0)
        start_idx(1, 1, False)

        zero16 = jnp.zeros((LANES,), f32)

        def zrow(i, carry):
            msg[0][i, pl.ds(0, LANES)] = zero16
            msg[0][i, pl.ds(LANES, LANES)] = zero16
            return carry

        lax.fori_loop(0, CH, zrow, 0)
        for off, ln in stripe_chunks:
            pltpu.sync_copy(
                msg[0].at[pl.ds(0, ln)],
                acc.at[pl.ds(sid * rows_per_tile + off, ln)])
        plsc.subcore_barrier()

        iota16 = lax.iota(jnp.int32, LANES)
        col_as = jnp.full((LANES,), 16, jnp.int32)
        col_ad = jnp.zeros((LANES,), jnp.int32)

        def pair(p, carry):
            for b in (0, 1):
                ob = 1 - b
                k = 2 * p + b

                @pl.when(k + 1 < n_chunks)
                def _():
                    wait_idx(ob)
                    unpack_idx(ob)
                    start_gather(ob)

                wait_gather(b)

                @pl.when(k >= 2)
                def _():
                    wait_scatter(b)

                for j in range(CH // LANES):
                    dsts[b][pl.ds(j * LANES, LANES)] = (
                        dstv[b][pl.ds(j * LANES, LANES)])

                @pl.when(k + 2 < n_chunks)
                def _():
                    start_idx(k + 2, b, False)

                rb, ab, mb = rows[b], adv[b], msg[b]
                onehot0 = jnp.where(iota16 == 0, 1.0, 0.0).astype(f32)

                def group(g, gcarry):
                    c_idx = g * LANES + iota16
                    asrc16 = plsc.load_gather(rb, [c_idx, col_as])
                    adst16 = plsc.load_gather(ab, [c_idx, col_ad])
                    e16 = asrc16 + adst16
                    e16 = jnp.where(e16 > 0.0, e16, NEG_SLOPE * e16)
                    w16 = jnp.exp(e16)
                    for l in range(LANES):
                        c = g * LANES + l
                        wb = _vbcast(w16, l)
                        mb[c, pl.ds(0, LANES)] = wb * rb[c, pl.ds(0, LANES)]
                        mb[c, pl.ds(LANES, LANES)] = wb * onehot0
                    return gcarry

                lax.fori_loop(0, CH // LANES, group, 0)
                pltpu.async_copy(mb, acc.at[dsts[b]], ssem[b], add=True)
            return carry

        lax.fori_loop(0, n_chunks // 2, pair, 0)
        wait_scatter(0)
        wait_scatter(1)
        plsc.subcore_barrier()
        for off, ln in stripe_chunks:
            sl = pl.ds(sid * rows_per_tile + off, ln)
            pltpu.sync_copy(acc.at[sl], out_hbm.at[cid, sl])

    return sc2


def _tc_final(acc2, b2, Np):
    R = 512
    f32 = jnp.float32

    def body(acc_ref, b2_ref, out_ref):
        a = acc_ref[0] + acc_ref[1]
        num = a[:, :16]
        den = a[:, 16:17]
        out_ref[...] = num / den + b2_ref[...]

    grid = ((Np + R - 1) // R,)
    return pl.pallas_call(
        body,
        grid=grid,
        in_specs=[
            pl.BlockSpec((2, R, 32), lambda i: (0, i, 0)),
            pl.BlockSpec((1, 16), lambda i: (0, 0)),
        ],
        out_specs=pl.BlockSpec((R, 16), lambda i: (i, 0)),
        out_shape=jax.ShapeDtypeStruct((Np, 16), f32),
    )(acc2, b2)


def kernel(x, edge_index, W1, a_src1, a_dst1, b1, W2, a_src2, a_dst2, b2):
    N, DIN = x.shape
    E = edge_index.shape[1]
    H1, F1 = a_src1.shape
    f32 = jnp.float32

    Np = _round_up(N + 1, NS)
    Etot = E + N
    Epad = _round_up(Etot, NT * CH * 2)

    loop = jnp.arange(N, dtype=edge_index.dtype)
    padv = jnp.full((Epad - Etot,), Np - 1, edge_index.dtype)
    src = jnp.concatenate([edge_index[0], loop, padv])
    dst = jnp.concatenate([edge_index[1], loop, padv])
    ei = jnp.concatenate(
        [src.reshape(Epad // CH, CH), dst.reshape(Epad // CH, CH)], axis=1)

    xp = jnp.zeros((Np, DIN), f32).at[:N].set(x)
    eye8 = jnp.eye(H1, dtype=f32)
    asrc_m = (eye8[:, None, :] * a_src1[:, :, None]).reshape(H1 * F1, H1)
    adst_m = (eye8[:, None, :] * a_dst1[:, :, None]).reshape(H1 * F1, H1)
    e8 = jnp.kron(jnp.eye(H1, dtype=f32), jnp.ones((1, F1), f32))
    perm = jnp.eye(H1 * F1, dtype=f32)

    tab1, adst1 = _tc_prep1(xp, W1, asrc_m, adst_m, perm, Np)
    acc1 = _sc_layer1(Np, Epad)(ei, tab1, adst1)

    tab2, adst2 = _tc_mid(acc1, b1.reshape(1, -1), W2,
                          a_src2.reshape(1, -1), a_dst2.reshape(1, -1),
                          e8, Np)

    acc2 = _sc_layer2(Np, Epad)(ei, tab2, adst2)

    out = _tc_final(acc2, b2.reshape(1, -1), Np)
    return out[:N]


# speedup vs baseline: 1.2409x; 1.0288x over previous
import functools

import jax
import jax.numpy as jnp
from jax import lax
from jax.experimental import pallas as pl
from jax.experimental.pallas import tpu as pltpu
from jax.experimental.pallas import tpu_sc as plsc

NEG_SLOPE = 0.2
NC = 2
NS = 16
NT = NC * NS
CH = 64
LANES = 16


def _round_up(v, m):
    return (v + m - 1) // m * m


def _vgather16(v, idx):
    dnums = lax.GatherDimensionNumbers(
        offset_dims=(), collapsed_slice_dims=(0,), start_index_map=(0,))
    return lax.gather(v, idx[:, None], dnums, (1,),
                      mode=lax.GatherScatterMode.PROMISE_IN_BOUNDS)


def _vbcast(v, lane):
    return _vgather16(v, jnp.full((LANES,), lane, jnp.int32))


def _tc_prep1(xp, W1, asrc_m, adst_m, Np):
    R = 512
    f32 = jnp.float32

    def body(x_ref, w_ref, as_ref, ad_ref, tab_ref, adst_ref):
        h = jnp.dot(x_ref[...], w_ref[...], preferred_element_type=f32)
        asrc = jnp.dot(h, as_ref[...], preferred_element_type=f32)
        adst = jnp.dot(h, ad_ref[...], preferred_element_type=f32)
        tab_ref[...] = jnp.concatenate([h, asrc, adst], axis=1)
        adst_ref[...] = jnp.concatenate(
            [adst, jnp.zeros((R, 8), f32)], axis=1)

    grid = ((Np + R - 1) // R,)
    return pl.pallas_call(
        body,
        grid=grid,
        in_specs=[
            pl.BlockSpec((R, 128), lambda i: (i, 0)),
            pl.BlockSpec((128, 128), lambda i: (0, 0)),
            pl.BlockSpec((128, 8), lambda i: (0, 0)),
            pl.BlockSpec((128, 8), lambda i: (0, 0)),
        ],
        out_specs=[
            pl.BlockSpec((R, 144), lambda i: (i, 0)),
            pl.BlockSpec((R, 16), lambda i: (i, 0)),
        ],
        out_shape=[
            jax.ShapeDtypeStruct((Np, 144), f32),
            jax.ShapeDtypeStruct((Np, 16), f32),
        ],
    )(xp, W1, asrc_m, adst_m)


def _sc_layer1(Np, Epad):
    per_tile = Epad // NT
    n_chunks = per_tile // CH
    rows_per_tile = Np // NS
    stripe_chunks = [(j * CH, CH) for j in range(rows_per_tile // CH)]
    if rows_per_tile % CH:
        stripe_chunks.append((rows_per_tile // CH * CH, rows_per_tile % CH))
    f32 = jnp.float32
    mesh = plsc.VectorSubcoreMesh(
        core_axis_name="c", subcore_axis_name="s", num_cores=NC,
        num_subcores=NS)

    assert n_chunks % 2 == 0

    @functools.partial(
        pl.kernel,
        out_type=jax.ShapeDtypeStruct((NC, Np, 144), f32),
        mesh=mesh,
        compiler_params=pltpu.CompilerParams(use_tc_tiling_on_sc=False, needs_layout_passes=False),
        scratch_types=[
            [pltpu.VMEM((2 * CH,), jnp.int32)] * 2,
            [pltpu.VMEM((CH,), jnp.int32)] * 2,
            [pltpu.VMEM((CH,), jnp.int32)] * 2,
            [pltpu.VMEM((CH,), jnp.int32)] * 2,
            [pltpu.VMEM((CH, 144), f32)] * 2,
            [pltpu.VMEM((CH, 16), f32)] * 2,
            [pltpu.VMEM((CH, 144), f32)] * 2,
            pltpu.VMEM_SHARED((Np, 144), f32),
            [pltpu.SemaphoreType.DMA] * 2,
            [pltpu.SemaphoreType.DMA] * 2,
            [pltpu.SemaphoreType.DMA] * 2,
            [pltpu.SemaphoreType.DMA] * 2,
        ],
    )
    def sc1(ei_hbm, tab_hbm, adst_hbm, out_hbm,
            eidx, srcv, dstv, dsts, rows, adv, msg, acc,
            isem, gsr, gsa, ssem):
        cid = lax.axis_index("c")
        sid = lax.axis_index("s")
        wid = sid * NC + cid
        cbase = wid * n_chunks

        def start_idx(k, b, sync):
            if sync:
                pltpu.sync_copy(ei_hbm.at[cbase + k], eidx[b])
            else:
                pltpu.async_copy(ei_hbm.at[cbase + k], eidx[b], isem[b])

        def wait_idx(b):
            pltpu.make_async_copy(
                ei_hbm.at[0], eidx[b], isem[b]).wait()

        def unpack_idx(b):
            for j in range(CH // LANES):
                srcv[b][pl.ds(j * LANES, LANES)] = (
                    eidx[b][pl.ds(j * LANES, LANES)])
                dstv[b][pl.ds(j * LANES, LANES)] = (
                    eidx[b][pl.ds(CH + j * LANES, LANES)])

        def start_gather(b):
            pltpu.async_copy(tab_hbm.at[srcv[b]], rows[b], gsr[b])
            pltpu.async_copy(adst_hbm.at[dstv[b]], adv[b], gsa[b])

        def wait_gather(b):
            pltpu.make_async_copy(
                tab_hbm.at[srcv[b]], rows[b], gsr[b]).wait()
            pltpu.make_async_copy(
                adst_hbm.at[dstv[b]], adv[b], gsa[b]).wait()

        def wait_scatter(b):
            pltpu.make_async_copy(
                msg[b], acc.at[dsts[b]], ssem[b]).wait()

        start_idx(0, 0, True)
        unpack_idx(0)
        start_gather(0)
        start_idx(1, 1, False)

        zero16 = jnp.zeros((LANES,), f32)

        def zrow(i, carry):
            for j in range(144 // LANES):
                msg[0][i, pl.ds(j * LANES, LANES)] = zero16
            return carry

        lax.fori_loop(0, CH, zrow, 0)
        for off, ln in stripe_chunks:
            pltpu.sync_copy(
                msg[0].at[pl.ds(0, ln)],
                acc.at[pl.ds(sid * rows_per_tile + off, ln)])
        plsc.subcore_barrier()

        def pair(p, carry):
            for b in (0, 1):
                ob = 1 - b
                k = 2 * p + b

                @pl.when(k + 1 < n_chunks)
                def _():
                    wait_idx(ob)
                    unpack_idx(ob)
                    start_gather(ob)

                wait_gather(b)

                @pl.when(k >= 2)
                def _():
                    wait_scatter(b)

                for j in range(CH // LANES):
                    dsts[b][pl.ds(j * LANES, LANES)] = (
                        dstv[b][pl.ds(j * LANES, LANES)])

                @pl.when(k + 2 < n_chunks)
                def _():
                    start_idx(k + 2, b, False)

                rb, ab, mb = rows[b], adv[b], msg[b]

                def _edge(q, ecarry):
                    for u in range(2):
                        c = 2 * q + u
                        e16 = rb[c, pl.ds(128, LANES)] + ab[c, :]
                        e16 = jnp.where(e16 > 0.0, e16, NEG_SLOPE * e16)
                        w16 = jnp.exp(e16)
                        mb[c, pl.ds(128, LANES)] = w16
                        for h in range(8):
                            wb = _vbcast(w16, h)
                            mb[c, pl.ds(h * LANES, LANES)] = (
                                wb * rb[c, pl.ds(h * LANES, LANES)])
                    return ecarry

                lax.fori_loop(0, CH // 2, _edge, 0)

                pltpu.async_copy(mb, acc.at[dsts[b]], ssem[b], add=True)
            return carry

        lax.fori_loop(0, n_chunks // 2, pair, 0)
        wait_scatter(0)
        wait_scatter(1)
        plsc.subcore_barrier()
        for off, ln in stripe_chunks:
            sl = pl.ds(sid * rows_per_tile + off, ln)
            pltpu.sync_copy(acc.at[sl], out_hbm.at[cid, sl])

    return sc1


def _tc_mid(acc1, b1, W2, asv2, adv2, e8, Np):
    R = 512
    f32 = jnp.float32

    def body(acc_ref, b1_ref, w2_ref, as_ref, ad_ref, e8_ref,
             tab_ref, adst_ref):
        a = acc_ref[0] + acc_ref[1]
        num = a[:, :128]
        den8 = a[:, 128:136]
        den = jnp.dot(den8, e8_ref[...], preferred_element_type=f32)
        pre = num / den + b1_ref[...]
        out1 = jnp.where(pre > 0.0, pre, jnp.exp(pre) - 1.0)
        h2 = jnp.dot(out1, w2_ref[...], preferred_element_type=f32)
        asrc2 = jnp.sum(h2 * as_ref[...], axis=1, keepdims=True)
        adst2 = jnp.sum(h2 * ad_ref[...], axis=1, keepdims=True)
        pad = jnp.zeros((R, 14), f32)
        tab_ref[...] = jnp.concatenate([h2, asrc2, adst2, pad], axis=1)
        adst_ref[...] = jnp.concatenate(
            [adst2, jnp.zeros((R, 7), f32)], axis=1)

    grid = ((Np + R - 1) // R,)
    return pl.pallas_call(
        body,
        grid=grid,
        in_specs=[
            pl.BlockSpec((2, R, 144), lambda i: (0, i, 0)),
            pl.BlockSpec((1, 128), lambda i: (0, 0)),
            pl.BlockSpec((128, 16), lambda i: (0, 0)),
            pl.BlockSpec((1, 16), lambda i: (0, 0)),
            pl.BlockSpec((1, 16), lambda i: (0, 0)),
            pl.BlockSpec((8, 128), lambda i: (0, 0)),
        ],
        out_specs=[
            pl.BlockSpec((R, 32), lambda i: (i, 0)),
            pl.BlockSpec((R, 8), lambda i: (i, 0)),
        ],
        out_shape=[
            jax.ShapeDtypeStruct((Np, 32), f32),
            jax.ShapeDtypeStruct((Np, 8), f32),
        ],
    )(acc1, b1, W2, asv2, adv2, e8)


def _sc_layer2(Np, Epad):
    per_tile = Epad // NT
    n_chunks = per_tile // CH
    rows_per_tile = Np // NS
    stripe_chunks = [(j * CH, CH) for j in range(rows_per_tile // CH)]
    if rows_per_tile % CH:
        stripe_chunks.append((rows_per_tile // CH * CH, rows_per_tile % CH))
    f32 = jnp.float32
    mesh = plsc.VectorSubcoreMesh(
        core_axis_name="c", subcore_axis_name="s", num_cores=NC,
        num_subcores=NS)

    assert n_chunks % 2 == 0

    @functools.partial(
        pl.kernel,
        out_type=jax.ShapeDtypeStruct((NC, Np, 32), f32),
        mesh=mesh,
        compiler_params=pltpu.CompilerParams(use_tc_tiling_on_sc=False, needs_layout_passes=False),
        scratch_types=[
            [pltpu.VMEM((2 * CH,), jnp.int32)] * 2,
            [pltpu.VMEM((CH,), jnp.int32)] * 2,
            [pltpu.VMEM((CH,), jnp.int32)] * 2,
            [pltpu.VMEM((CH,), jnp.int32)] * 2,
            [pltpu.VMEM((CH, 32), f32)] * 2,
            [pltpu.VMEM((CH, 8), f32)] * 2,
            [pltpu.VMEM((CH, 32), f32)] * 2,
            pltpu.VMEM_SHARED((Np, 32), f32),
            [pltpu.SemaphoreType.DMA] * 2,
            [pltpu.SemaphoreType.DMA] * 2,
            [pltpu.SemaphoreType.DMA] * 2,
            [pltpu.SemaphoreType.DMA] * 2,
        ],
    )
    def sc2(ei_hbm, tab_hbm, adst_hbm, out_hbm,
            eidx, srcv, dstv, dsts, rows, adv, msg, acc,
            isem, gsr, gsa, ssem):
        cid = lax.axis_index("c")
        sid = lax.axis_index("s")
        wid = sid * NC + cid
        cbase = wid * n_chunks

        def start_idx(k, b, sync):
            if sync:
                pltpu.sync_copy(ei_hbm.at[cbase + k], eidx[b])
            else:
                pltpu.async_copy(ei_hbm.at[cbase + k], eidx[b], isem[b])

        def wait_idx(b):
            pltpu.make_async_copy(
                ei_hbm.at[0], eidx[b], isem[b]).wait()

        def unpack_idx(b):
            for j in range(CH // LANES):
                srcv[b][pl.ds(j * LANES, LANES)] = (
                    eidx[b][pl.ds(j * LANES, LANES)])
                dstv[b][pl.ds(j * LANES, LANES)] = (
                    eidx[b][pl.ds(CH + j * LANES, LANES)])

        def start_gather(b):
            pltpu.async_copy(tab_hbm.at[srcv[b]], rows[b], gsr[b])
            pltpu.async_copy(adst_hbm.at[dstv[b]], adv[b], gsa[b])

        def wait_gather(b):
            pltpu.make_async_copy(
                tab_hbm.at[srcv[b]], rows[b], gsr[b]).wait()
            pltpu.make_async_copy(
                adst_hbm.at[dstv[b]], adv[b], gsa[b]).wait()

        def wait_scatter(b):
            pltpu.make_async_copy(
                msg[b], acc.at[dsts[b]], ssem[b]).wait()

        start_idx(0, 0, True)
        unpack_idx(0)
        start_gather(0)
        start_idx(1, 1, False)

        zero16 = jnp.zeros((LANES,), f32)

        def zrow(i, carry):
            msg[0][i, pl.ds(0, LANES)] = zero16
            msg[0][i, pl.ds(LANES, LANES)] = zero16
            return carry

        lax.fori_loop(0, CH, zrow, 0)
        for off, ln in stripe_chunks:
            pltpu.sync_copy(
                msg[0].at[pl.ds(0, ln)],
                acc.at[pl.ds(sid * rows_per_tile + off, ln)])
        plsc.subcore_barrier()

        iota16 = lax.iota(jnp.int32, LANES)
        col_as = jnp.full((LANES,), 16, jnp.int32)
        col_ad = jnp.zeros((LANES,), jnp.int32)

        def pair(p, carry):
            for b in (0, 1):
                ob = 1 - b
                k = 2 * p + b

                @pl.when(k + 1 < n_chunks)
                def _():
                    wait_idx(ob)
                    unpack_idx(ob)
                    start_gather(ob)

                wait_gather(b)

                @pl.when(k >= 2)
                def _():
                    wait_scatter(b)

                for j in range(CH // LANES):
                    dsts[b][pl.ds(j * LANES, LANES)] = (
                        dstv[b][pl.ds(j * LANES, LANES)])

                @pl.when(k + 2 < n_chunks)
                def _():
                    start_idx(k + 2, b, False)

                rb, ab, mb = rows[b], adv[b], msg[b]
                onehot0 = jnp.where(iota16 == 0, 1.0, 0.0).astype(f32)

                def group(g, gcarry):
                    c_idx = g * LANES + iota16
                    asrc16 = plsc.load_gather(rb, [c_idx, col_as])
                    adst16 = plsc.load_gather(ab, [c_idx, col_ad])
                    e16 = asrc16 + adst16
                    e16 = jnp.where(e16 > 0.0, e16, NEG_SLOPE * e16)
                    w16 = jnp.exp(e16)
                    for l in range(LANES):
                        c = g * LANES + l
                        wb = _vbcast(w16, l)
                        mb[c, pl.ds(0, LANES)] = wb * rb[c, pl.ds(0, LANES)]
                        mb[c, pl.ds(LANES, LANES)] = wb * onehot0
                    return gcarry

                lax.fori_loop(0, CH // LANES, group, 0)
                pltpu.async_copy(mb, acc.at[dsts[b]], ssem[b], add=True)
            return carry

        lax.fori_loop(0, n_chunks // 2, pair, 0)
        wait_scatter(0)
        wait_scatter(1)
        plsc.subcore_barrier()
        for off, ln in stripe_chunks:
            sl = pl.ds(sid * rows_per_tile + off, ln)
            pltpu.sync_copy(acc.at[sl], out_hbm.at[cid, sl])

    return sc2


def _tc_final(acc2, b2, Np):
    R = 512
    f32 = jnp.float32

    def body(acc_ref, b2_ref, out_ref):
        a = acc_ref[0] + acc_ref[1]
        num = a[:, :16]
        den = a[:, 16:17]
        out_ref[...] = num / den + b2_ref[...]

    grid = ((Np + R - 1) // R,)
    return pl.pallas_call(
        body,
        grid=grid,
        in_specs=[
            pl.BlockSpec((2, R, 32), lambda i: (0, i, 0)),
            pl.BlockSpec((1, 16), lambda i: (0, 0)),
        ],
        out_specs=pl.BlockSpec((R, 16), lambda i: (i, 0)),
        out_shape=jax.ShapeDtypeStruct((Np, 16), f32),
    )(acc2, b2)


def kernel(x, edge_index, W1, a_src1, a_dst1, b1, W2, a_src2, a_dst2, b2):
    N, DIN = x.shape
    E = edge_index.shape[1]
    H1, F1 = a_src1.shape
    f32 = jnp.float32

    Np = _round_up(N + 1, NS)
    Etot = E + N
    Epad = _round_up(Etot, NT * CH * 2)

    loop = jnp.arange(N, dtype=edge_index.dtype)
    padv = jnp.full((Epad - Etot,), Np - 1, edge_index.dtype)
    src = jnp.concatenate([edge_index[0], loop, padv])
    dst = jnp.concatenate([edge_index[1], loop, padv])
    ei = jnp.concatenate(
        [src.reshape(Epad // CH, CH), dst.reshape(Epad // CH, CH)], axis=1)

    xp = jnp.zeros((Np, DIN), f32).at[:N].set(x)
    eye8 = jnp.eye(H1, dtype=f32)
    asrc_m = (eye8[:, None, :] * a_src1[:, :, None]).reshape(H1 * F1, H1)
    adst_m = (eye8[:, None, :] * a_dst1[:, :, None]).reshape(H1 * F1, H1)
    e8 = jnp.kron(jnp.eye(H1, dtype=f32), jnp.ones((1, F1), f32))

    tab1, adst1 = _tc_prep1(xp, W1, asrc_m, adst_m, Np)
    acc1 = _sc_layer1(Np, Epad)(ei, tab1, adst1)

    tab2, adst2 = _tc_mid(acc1, b1.reshape(1, -1), W2,
                          a_src2.reshape(1, -1), a_dst2.reshape(1, -1),
                          e8, Np)

    acc2 = _sc_layer2(Np, Epad)(ei, tab2, adst2)

    out = _tc_final(acc2, b2.reshape(1, -1), Np)
    return out[:N]
